# baseline (speedup 1.0000x reference)
import functools

import jax
import jax.numpy as jnp
from jax import lax
from jax.experimental import pallas as pl
from jax.experimental.pallas import tpu as pltpu
from jax.experimental.pallas import tpu_sc as plsc

DIM = 2048
E = 64
EPS = 1e-06


def _gate_kernel(x_ref, w_ref, b_ref, hit_ref, p0_ref, hit_s, p0_s,
                 *, n_st, n_b):
    bi = pl.program_id(0)
    st = pl.program_id(1)

    xb = x_ref[0]
    w = w_ref[...]
    logits = jax.lax.dot_general(
        xb, w, (((1,), (1,)), ((), ())),
        preferred_element_type=jnp.float32) + b_ref[0]

    rowmax = jnp.max(logits, axis=1, keepdims=True)
    iota = jax.lax.broadcasted_iota(jnp.int32, logits.shape, 1)
    first = jnp.min(jnp.where(logits == rowmax, iota, E), axis=1,
                    keepdims=True)
    onehot = (iota == first).astype(jnp.float32)
    hit_part = jnp.max(onehot, axis=0, keepdims=True)

    @pl.when(st == 0)
    def _():
        hit_s[pl.ds(bi, 1), :] = hit_part
        rows = logits[:E]
        m = jnp.max(rows, axis=1, keepdims=True)
        ex = jnp.exp(rows - m)
        se = jnp.sum(ex, axis=1, keepdims=True)
        p0_s[pl.ds(bi, 1), :] = (ex[:, :1] / se).reshape(1, E)

    @pl.when(st != 0)
    def _():
        hit_s[pl.ds(bi, 1), :] = jnp.maximum(hit_s[pl.ds(bi, 1), :], hit_part)

    @pl.when(jnp.logical_and(bi == n_b - 1, st == n_st - 1))
    def _():
        hit_ref[...] = hit_s[...]
        p0_ref[...] = p0_s[...]


def _sc_finalize(hit_hbm, p0_hbm, vals_hbm, loss_hbm, hit_v, p0_v, vals_v,
                 loss_v, tmp_v, *, n_b, seq, cap):
    L = 16
    nchunk = n_b * E // L
    jc = E // L
    lane = lax.broadcasted_iota(jnp.int32, (L,), 0)

    dnums = lax.GatherDimensionNumbers(
        offset_dims=(), collapsed_slice_dims=(0,), start_index_map=(0,))

    def lane_allsum(v):
        for sh in (8, 4, 2, 1):
            idx = jnp.bitwise_xor(lane, sh)
            shuf = lax.gather(v, idx[:, None], dnums, (1,),
                              mode=lax.GatherScatterMode.PROMISE_IN_BOUNDS)
            v = v + shuf
        return v

    @pl.when(jnp.logical_and(lax.axis_index("c") == 0,
                             lax.axis_index("s") == 0))
    def _():
        pltpu.sync_copy(hit_hbm, hit_v)
        pltpu.sync_copy(p0_hbm, p0_v)

        m_chunks = [hit_v[pl.ds(k * L, L)] * p0_v[pl.ds(k * L, L)]
                    for k in range(nchunk)]
        d_chunks = []
        for j in range(jc):
            d = m_chunks[j]
            for b in range(1, n_b):
                d = d + m_chunks[b * jc + j]
            d_chunks.append(d + EPS)
        v_chunks = [m_chunks[k] / d_chunks[k % jc] * cap
                    for k in range(nchunk)]
        for k in range(nchunk):
            vals_v[pl.ds(k * L, L)] = v_chunks[k]

        zero = jnp.zeros((L,), jnp.float32)
        a_i = a_i2 = a_l = a_l2 = zero
        for j in range(jc):
            imp = v_chunks[j]
            ld = jnp.where(v_chunks[j] > 0.0, 1.0, 0.0)
            for b in range(1, n_b):
                imp = imp + v_chunks[b * jc + j]
                ld = ld + jnp.where(v_chunks[b * jc + j] > 0.0, 1.0, 0.0)
            a_i = a_i + imp
            a_i2 = a_i2 + imp * imp
            a_l = a_l + ld
            a_l2 = a_l2 + ld * ld

        s1i = lane_allsum(a_i)
        s2i = lane_allsum(a_i2)
        s1l = lane_allsum(a_l)
        s2l = lane_allsum(a_l2)

        n = float(seq * E)
        def cv2(s1, s2):
            m_ = s1 / n
            var = (s2 - n * m_ * m_) / (n - 1.0)
            return var / (m_ * m_ + 1e-10)

        loss = cv2(s1i, s2i) + cv2(s1l, s2l)
        loss_v[...] = jnp.where(lane == 0, loss, zero)

        pltpu.sync_copy(vals_v, vals_hbm)
        pltpu.sync_copy(loss_v, loss_hbm)


@jax.jit
def kernel(x, W, b):
    B, S, D = x.shape
    ne = W.shape[0]
    cap = float(int(1.0 * B))
    TS = 1024
    n_st = S // TS
    grid = (B, n_st)

    hit, p0 = pl.pallas_call(
        functools.partial(_gate_kernel, n_st=n_st, n_b=B),
        grid=grid,
        in_specs=[
            pl.BlockSpec((1, TS, D), lambda bi, st: (bi, st, 0)),
            pl.BlockSpec((ne, D), lambda bi, st: (0, 0)),
            pl.BlockSpec((1, ne), lambda bi, st: (0, 0)),
        ],
        out_specs=[
            pl.BlockSpec((B, ne), lambda bi, st: (0, 0)),
            pl.BlockSpec((B, ne), lambda bi, st: (0, 0)),
        ],
        out_shape=[
            jax.ShapeDtypeStruct((B, ne), jnp.float32),
            jax.ShapeDtypeStruct((B, ne), jnp.float32),
        ],
        scratch_shapes=[
            pltpu.VMEM((B, ne), jnp.float32),
            pltpu.VMEM((B, ne), jnp.float32),
        ],
    )(x, W, b.reshape(1, ne))

    mesh = plsc.VectorSubcoreMesh(core_axis_name="c", subcore_axis_name="s")
    sc_fin = pl.kernel(
        functools.partial(_sc_finalize, n_b=B, seq=S, cap=cap),
        mesh=mesh,
        out_type=[
            jax.ShapeDtypeStruct((B * ne,), jnp.float32),
            jax.ShapeDtypeStruct((16,), jnp.float32),
        ],
        scratch_types=[
            pltpu.VMEM((B * ne,), jnp.float32),
            pltpu.VMEM((B * ne,), jnp.float32),
            pltpu.VMEM((B * ne,), jnp.float32),
            pltpu.VMEM((16,), jnp.float32),
            pltpu.VMEM((16,), jnp.float32),
        ],
    )
    vals_flat, loss_vec = sc_fin(hit.reshape(B * ne), p0.reshape(B * ne))
    vals = vals_flat.reshape(B, ne)

    gs = jnp.zeros((B, S, ne), jnp.float32).at[:, :ne, 0].set(vals)
    return gs, loss_vec[0]


# speedup vs baseline: 1.7269x; 1.7269x over previous
import functools

import jax
import jax.numpy as jnp
from jax.experimental import pallas as pl
import jax.experimental.pallas.tpu as pltpu

DIM = 2048
E = 64
EPS = 1e-06


def _router_kernel(x_ref, w_ref, b_ref, vals_ref, loss_ref, hit_s, p0_s,
                   *, n_st, n_b, seq, cap):
    bi = pl.program_id(0)
    st = pl.program_id(1)

    xb = x_ref[0]
    w = w_ref[...]
    logits = jax.lax.dot_general(
        xb, w, (((1,), (1,)), ((), ())),
        preferred_element_type=jnp.float32) + b_ref[0]

    rowmax = jnp.max(logits, axis=1, keepdims=True)
    iota = jax.lax.broadcasted_iota(jnp.int32, logits.shape, 1)
    first = jnp.min(jnp.where(logits == rowmax, iota, E), axis=1,
                    keepdims=True)
    onehot = (iota == first).astype(jnp.float32)
    hit_part = jnp.max(onehot, axis=0, keepdims=True)

    @pl.when(st == 0)
    def _():
        hit_s[pl.ds(bi, 1), :] = hit_part
        rows = logits[:E]
        m = jnp.max(rows, axis=1, keepdims=True)
        ex = jnp.exp(rows - m)
        se = jnp.sum(ex, axis=1, keepdims=True)
        p0_s[pl.ds(bi, 1), :] = (ex[:, :1] / se).reshape(1, E)

    @pl.when(st != 0)
    def _():
        hit_s[pl.ds(bi, 1), :] = jnp.maximum(hit_s[pl.ds(bi, 1), :], hit_part)

    @pl.when(jnp.logical_and(bi == n_b - 1, st == n_st - 1))
    def _():
        hit = hit_s[...]
        p0 = p0_s[...]
        masked = p0 * hit
        denom = jnp.sum(masked, axis=0, keepdims=True) + EPS
        vals = masked / denom * cap
        vals_ref[...] = vals
        imp = jnp.sum(vals, axis=0)
        load = jnp.sum((vals > 0).astype(jnp.float32), axis=0)

        n = float(seq * E)
        def cv2(v):
            s1 = jnp.sum(v)
            s2 = jnp.sum(v * v)
            m_ = s1 / n
            var = (s2 - n * m_ * m_) / (n - 1.0)
            return var / (m_ * m_ + 1e-10)

        loss_ref[...] = (cv2(imp) + cv2(load)).reshape(1, 1)


@jax.jit
def kernel(x, W, b):
    B, S, D = x.shape
    ne = W.shape[0]
    cap = float(int(1.0 * B))
    TS = 1024
    n_st = S // TS
    grid = (B, n_st)

    vals, loss = pl.pallas_call(
        functools.partial(_router_kernel, n_st=n_st, n_b=B, seq=S, cap=cap),
        grid=grid,
        in_specs=[
            pl.BlockSpec((1, TS, D), lambda bi, st: (bi, st, 0)),
            pl.BlockSpec((ne, D), lambda bi, st: (0, 0)),
            pl.BlockSpec((1, ne), lambda bi, st: (0, 0)),
        ],
        out_specs=[
            pl.BlockSpec((B, ne), lambda bi, st: (0, 0)),
            pl.BlockSpec((1, 1), lambda bi, st: (0, 0)),
        ],
        out_shape=[
            jax.ShapeDtypeStruct((B, ne), jnp.float32),
            jax.ShapeDtypeStruct((1, 1), jnp.float32),
        ],
        scratch_shapes=[
            pltpu.VMEM((B, ne), jnp.float32),
            pltpu.VMEM((B, ne), jnp.float32),
        ],
    )(x, W, b.reshape(1, ne))

    gs = jax.lax.pad(vals[:, :, None], jnp.float32(0.0),
                     ((0, 0, 0), (0, S - ne, 0), (0, ne - 1, 0)))
    return gs, loss[0, 0]
